# Initial kernel scaffold; baseline (speedup 1.0000x reference)
import jax
import jax.numpy as jnp
from jax.experimental import pallas as pl


def kernel(x, ref_emb, We1, be1, We2, be2, Wd1, bd1, Wd2, bd2):
    raise NotImplementedError("write your pallas kernel here")


# speedup vs baseline: 83.2321x; 83.2321x over previous
import functools

import jax
import jax.numpy as jnp
from jax import lax
from jax.experimental import pallas as pl
from jax.experimental.pallas import tpu as pltpu
from jax.experimental.pallas import tpu_sc as plsc

B = 1024
D = 768
H = 512
L = 128
K = 10
BLK = 128
NBLK = B // BLK
NPAIR = K * K
HP = jax.lax.Precision.HIGHEST


def _k1_body(xb_ref, xf_ref, we1_ref, be1_ref, we2_ref, be2_ref,
             wd1_ref, bd1_ref, wd2_ref, bd2_ref,
             z_ref, p_ref, pair_ref, rec_ref):
    xb = xb_ref[...]
    xf = xf_ref[...]
    h1 = jnp.maximum(jnp.dot(xb, we1_ref[...], precision=HP) + be1_ref[...], 0.0)
    zb = jnp.dot(h1, we2_ref[...], precision=HP) + be2_ref[...]
    h2 = jnp.maximum(jnp.dot(zb, wd1_ref[...], precision=HP) + bd1_ref[...], 0.0)
    xr = jnp.dot(h2, wd2_ref[...], precision=HP) + bd2_ref[...]
    z_ref[...] = zb
    rec_part = jnp.sum((xr - xb) ** 2)
    rec_ref[...] = jnp.full((1, 1, 128), rec_part, jnp.float32)
    pb = lax.dot_general(xb, xf, (((1,), (1,)), ((), ())), precision=HP)
    p_ref[...] = pb
    xsq_b = jnp.sum(xb * xb, axis=1, keepdims=True)
    ones_row = jnp.ones((1, D), jnp.float32)
    xsq_all = lax.dot_general(ones_row, xf * xf, (((1,), (1,)), ((), ())),
                              precision=HP)
    d2 = jnp.maximum(xsq_b + xsq_all - 2.0 * pb, 1e-12)
    col = lax.broadcasted_iota(jnp.int32, (BLK, B), 1)
    w = d2
    idxs = []
    for _ in range(K + 1):
        m = jnp.min(w, axis=1, keepdims=True)
        am = jnp.min(jnp.where(w == m, col, jnp.int32(2**30)), axis=1,
                     keepdims=True)
        idxs.append(am)
        w = jnp.where(col == am, jnp.float32(jnp.inf), w)
    knn = jnp.concatenate(idxs[1:], axis=1)
    knnb = knn * B
    parts = [knnb[:, a:a + 1] + knn for a in range(K)]
    pair_ref[...] = jnp.concatenate(parts, axis=1)


def _run_k1(x, ref_unused, We1, be1, We2, be2, Wd1, bd1, Wd2, bd2,
            interpret=False):
    grid = (NBLK,)
    out_shapes = (
        jax.ShapeDtypeStruct((B, L), jnp.float32),
        jax.ShapeDtypeStruct((B, B), jnp.float32),
        jax.ShapeDtypeStruct((B, NPAIR), jnp.int32),
        jax.ShapeDtypeStruct((NBLK, 1, 128), jnp.float32),
    )
    full = lambda shape: pl.BlockSpec(shape, lambda i: (0,) * len(shape))
    in_specs = [
        pl.BlockSpec((BLK, D), lambda i: (i, 0)),
        full((B, D)),
        full((D, H)), full((1, H)),
        full((H, L)), full((1, L)),
        full((L, H)), full((1, H)),
        full((H, D)), full((1, D)),
    ]
    out_specs = (
        pl.BlockSpec((BLK, L), lambda i: (i, 0)),
        pl.BlockSpec((BLK, B), lambda i: (i, 0)),
        pl.BlockSpec((BLK, NPAIR), lambda i: (i, 0)),
        pl.BlockSpec((1, 1, 128), lambda i: (i, 0, 0)),
    )
    return pl.pallas_call(
        _k1_body, grid=grid, in_specs=in_specs, out_specs=out_specs,
        out_shape=out_shapes, interpret=interpret,
    )(x, x, We1, be1.reshape(1, H), We2, be2.reshape(1, L),
      Wd1, bd1.reshape(1, H), Wd2, bd2.reshape(1, D))


def _sc_gather(p_flat, idx_flat):
    n = idx_flat.shape[0]
    info = plsc.get_sparse_core_info()
    nw = info.num_cores * info.num_subcores
    chunk = n // nw
    mesh = plsc.VectorSubcoreMesh(core_axis_name="c", subcore_axis_name="s")

    @functools.partial(
        pl.kernel, mesh=mesh,
        out_type=jax.ShapeDtypeStruct((n,), jnp.float32),
        scratch_types=[
            pltpu.VMEM((chunk,), jnp.int32),
            pltpu.VMEM((chunk,), jnp.float32),
            pltpu.SemaphoreType.DMA,
        ],
    )
    def gather_k(p_hbm, idx_hbm, out_hbm, idx_v, val_v, sem):
        wid = lax.axis_index("s") * info.num_cores + lax.axis_index("c")
        base = wid * chunk
        pltpu.sync_copy(idx_hbm.at[pl.ds(base, chunk)], idx_v)
        pltpu.async_copy(p_hbm.at[idx_v], val_v, sem).wait()
        pltpu.sync_copy(val_v, out_hbm.at[pl.ds(base, chunk)])

    return gather_k(p_flat, idx_flat)


def _k3_body(ps_ref, curv_ref):
    pt = ps_ref[...]
    ra = [jnp.sum(pt[K * a:K * a + K, :], axis=0, keepdims=True)
          for a in range(K)]
    s = jnp.sum(pt, axis=0, keepdims=True)
    sc2 = s / (K * K)
    g = {}
    for a in range(K):
        for b in range(a, K):
            r = a * K + b
            g[(a, b)] = pt[r:r + 1, :] - ra[a] / K - ra[b] / K + sc2

    def sweep(_, g):
        def get(a, b):
            return g[(a, b)] if a <= b else g[(b, a)]

        for p in range(K - 1):
            for q in range(p + 1, K):
                app = g[(p, p)]
                apq = g[(p, q)]
                aqq = g[(q, q)]
                small = jnp.abs(apq) < 1e-30
                safe = jnp.where(small, jnp.float32(1.0), apq)
                tau = (aqq - app) / (2.0 * safe)
                t = jnp.sign(tau) / (jnp.abs(tau) + jnp.sqrt(1.0 + tau * tau))
                c = 1.0 / jnp.sqrt(1.0 + t * t)
                sn = t * c
                t = jnp.where(small, jnp.float32(0.0), t)
                c = jnp.where(small, jnp.float32(1.0), c)
                sn = jnp.where(small, jnp.float32(0.0), sn)
                for k in range(K):
                    if k == p or k == q:
                        continue
                    akp = get(k, p)
                    akq = get(k, q)
                    nkp = c * akp - sn * akq
                    nkq = sn * akp + c * akq
                    if k <= p:
                        g[(k, p)] = nkp
                    else:
                        g[(p, k)] = nkp
                    if k <= q:
                        g[(k, q)] = nkq
                    else:
                        g[(q, k)] = nkq
                g[(p, p)] = app - t * apq
                g[(q, q)] = aqq + t * apq
                g[(p, q)] = jnp.zeros_like(apq)
        return g

    g = lax.fori_loop(0, 6, sweep, g)
    sv = [jnp.sqrt(jnp.maximum(g[(k, k)], 0.0)) for k in range(K)]
    smax = functools.reduce(jnp.maximum, sv)
    ssum = functools.reduce(jnp.add, sv)
    curv = 1.0 - smax / (ssum + 1e-8)
    curv_ref[...] = curv[None]


def _run_k3(psub_t, interpret=False):
    return pl.pallas_call(
        _k3_body, grid=(NBLK,),
        in_specs=[pl.BlockSpec((NPAIR, BLK), lambda i: (0, i))],
        out_specs=pl.BlockSpec((1, 1, BLK), lambda i: (i, 0, 0)),
        out_shape=jax.ShapeDtypeStruct((NBLK, 1, BLK), jnp.float32),
        interpret=interpret,
    )(psub_t)


def _k4_body(zb_ref, zf_ref, rb_ref, rf_ref, ca_ref, cb_ref, out_ref):
    i = pl.program_id(0)
    zb = zb_ref[...]
    zf = zf_ref[...]
    rb = rb_ref[...]
    rf = rf_ref[...]

    def dist(ab, af):
        p = lax.dot_general(ab, af, (((1,), (1,)), ((), ())), precision=HP)
        sq_b = jnp.sum(ab * ab, axis=1, keepdims=True)
        ones_row = jnp.ones((1, L), jnp.float32)
        sq_all = lax.dot_general(ones_row, af * af, (((1,), (1,)), ((), ())),
                                 precision=HP)
        return jnp.sqrt(jnp.maximum(sq_b + sq_all - 2.0 * p, 1e-12))

    zd = dist(zb, zf)
    rd = dist(rb, rf)
    wgt = jnp.maximum(1.0 - jnp.maximum(ca_ref[...], cb_ref[...]), 0.1)
    row = lax.broadcasted_iota(jnp.int32, (BLK, B), 0) + i * BLK
    colg = lax.broadcasted_iota(jnp.int32, (BLK, B), 1)
    mw = jnp.where(colg > row, wgt, 0.0)
    sa = jnp.sum(mw * zd * zd)
    sb = jnp.sum(mw * zd * rd)
    sc = jnp.sum(mw * rd * rd)
    mz = jnp.max(zd)
    mr = jnp.max(rd)
    lane = lax.broadcasted_iota(jnp.int32, (1, 1, 128), 2)
    out_ref[...] = (sa * (lane == 0) + sb * (lane == 1) + sc * (lane == 2)
                    + mz * (lane == 3) + mr * (lane == 4)).astype(jnp.float32)


def _run_k4(z, ref_emb, curv_col, curv_row, interpret=False):
    full = lambda shape: pl.BlockSpec(shape, lambda i: (0,) * len(shape))
    return pl.pallas_call(
        _k4_body, grid=(NBLK,),
        in_specs=[
            pl.BlockSpec((BLK, L), lambda i: (i, 0)), full((B, L)),
            pl.BlockSpec((BLK, L), lambda i: (i, 0)), full((B, L)),
            pl.BlockSpec((BLK, 1), lambda i: (i, 0)), full((1, B)),
        ],
        out_specs=pl.BlockSpec((1, 1, 128), lambda i: (i, 0, 0)),
        out_shape=jax.ShapeDtypeStruct((NBLK, 1, 128), jnp.float32),
        interpret=interpret,
    )(z, z, ref_emb, ref_emb, curv_col, curv_row)


def kernel(x, ref_emb, We1, be1, We2, be2, Wd1, bd1, Wd2, bd2):
    z, p, pairs, rec_parts = _run_k1(x, ref_emb, We1, be1, We2, be2,
                                     Wd1, bd1, Wd2, bd2)
    rec_loss = jnp.sum(rec_parts[:, 0, 0]) / (B * D)
    idx_t = jnp.transpose(pairs).reshape(-1)
    psub_t = _sc_gather(p.reshape(-1), idx_t).reshape(NPAIR, B)
    curv = _run_k3(psub_t)
    curv_flat = curv.reshape(B)
    acc = _run_k4(z, ref_emb, curv_flat.reshape(B, 1), curv_flat.reshape(1, B))
    a = jnp.sum(acc[:, 0, 0])
    bsum = jnp.sum(acc[:, 0, 1])
    c = jnp.sum(acc[:, 0, 2])
    mz = jnp.max(acc[:, 0, 3]) + 1e-8
    mr = jnp.max(acc[:, 0, 4]) + 1e-8
    npairs = B * (B - 1) / 2.0
    dist_loss = (a / (mz * mz) - 2.0 * bsum / (mz * mr) + c / (mr * mr)) / npairs
    total = rec_loss + dist_loss
    return total, rec_loss, dist_loss


# speedup vs baseline: 130.8770x; 1.5724x over previous
import functools

import jax
import jax.numpy as jnp
from jax import lax
from jax.experimental import pallas as pl
from jax.experimental.pallas import tpu as pltpu
from jax.experimental.pallas import tpu_sc as plsc

B = 1024
D = 768
H = 512
L = 128
K = 10
BLK = 128
NBLK = B // BLK
NPAIR = K * K
SWEEPS = 5
HP = jax.lax.Precision.HIGHEST


def _k1_body(xb_ref, xf_ref, we1_ref, be1_ref, we2_ref, be2_ref,
             wd1_ref, bd1_ref, wd2_ref, bd2_ref,
             z_ref, p_ref, pair_ref, rec_ref):
    xb = xb_ref[...]
    xf = xf_ref[...]
    h1 = jnp.maximum(jnp.dot(xb, we1_ref[...], precision=HP) + be1_ref[...], 0.0)
    zb = jnp.dot(h1, we2_ref[...], precision=HP) + be2_ref[...]
    h2 = jnp.maximum(jnp.dot(zb, wd1_ref[...], precision=HP) + bd1_ref[...], 0.0)
    xr = jnp.dot(h2, wd2_ref[...], precision=HP) + bd2_ref[...]
    z_ref[...] = zb
    rec_part = jnp.sum((xr - xb) ** 2)
    rec_ref[...] = jnp.full((1, 1, 128), rec_part, jnp.float32)
    pb = lax.dot_general(xf, xb, (((1,), (1,)), ((), ())), precision=HP)
    p_ref[...] = pb
    xsq_col = jnp.sum(xf * xf, axis=1, keepdims=True)
    ones_row = jnp.ones((1, D), jnp.float32)
    xsq_row = lax.dot_general(ones_row, xb * xb, (((1,), (1,)), ((), ())),
                              precision=HP)
    d2 = jnp.maximum(xsq_col + xsq_row - 2.0 * pb, 1e-12)
    row = lax.broadcasted_iota(jnp.int32, (B, BLK), 0)
    w = d2
    idxs = []
    for _ in range(K + 1):
        m = jnp.min(w, axis=0, keepdims=True)
        am = jnp.min(jnp.where(w == m, row, jnp.int32(2**30)), axis=0,
                     keepdims=True)
        idxs.append(am)
        w = jnp.where(row == am, jnp.float32(jnp.inf), w)
    knn_t = jnp.concatenate(idxs[1:], axis=0)
    knn_tb = knn_t * B
    parts = [knn_tb[a:a + 1, :] + knn_t for a in range(K)]
    pair_ref[...] = jnp.concatenate(parts, axis=0)


def _run_k1(x, We1, be1, We2, be2, Wd1, bd1, Wd2, bd2, interpret=False):
    grid = (NBLK,)
    out_shapes = (
        jax.ShapeDtypeStruct((B, L), jnp.float32),
        jax.ShapeDtypeStruct((B, B), jnp.float32),
        jax.ShapeDtypeStruct((NPAIR, B), jnp.int32),
        jax.ShapeDtypeStruct((NBLK, 1, 128), jnp.float32),
    )
    full = lambda shape: pl.BlockSpec(shape, lambda i: (0,) * len(shape))
    in_specs = [
        pl.BlockSpec((BLK, D), lambda i: (i, 0)),
        full((B, D)),
        full((D, H)), full((1, H)),
        full((H, L)), full((1, L)),
        full((L, H)), full((1, H)),
        full((H, D)), full((1, D)),
    ]
    out_specs = (
        pl.BlockSpec((BLK, L), lambda i: (i, 0)),
        pl.BlockSpec((B, BLK), lambda i: (0, i)),
        pl.BlockSpec((NPAIR, BLK), lambda i: (0, i)),
        pl.BlockSpec((1, 1, 128), lambda i: (i, 0, 0)),
    )
    return pl.pallas_call(
        _k1_body, grid=grid, in_specs=in_specs, out_specs=out_specs,
        out_shape=out_shapes, interpret=interpret,
    )(x, x, We1, be1.reshape(1, H), We2, be2.reshape(1, L),
      Wd1, bd1.reshape(1, H), Wd2, bd2.reshape(1, D))


def _sc_gather(p_flat, idx_flat):
    n = idx_flat.shape[0]
    info = plsc.get_sparse_core_info()
    nw = info.num_cores * info.num_subcores
    chunk = n // nw
    mesh = plsc.VectorSubcoreMesh(core_axis_name="c", subcore_axis_name="s")

    @functools.partial(
        pl.kernel, mesh=mesh,
        out_type=jax.ShapeDtypeStruct((n,), jnp.float32),
        scratch_types=[
            pltpu.VMEM((chunk,), jnp.int32),
            pltpu.VMEM((chunk,), jnp.float32),
            pltpu.SemaphoreType.DMA,
        ],
    )
    def gather_k(p_hbm, idx_hbm, out_hbm, idx_v, val_v, sem):
        wid = lax.axis_index("s") * info.num_cores + lax.axis_index("c")
        base = wid * chunk
        pltpu.sync_copy(idx_hbm.at[pl.ds(base, chunk)], idx_v)
        pltpu.async_copy(p_hbm.at[idx_v], val_v, sem).wait()
        pltpu.sync_copy(val_v, out_hbm.at[pl.ds(base, chunk)])

    return gather_k(p_flat, idx_flat)


def _k3_body(ps_ref, curv_ref):
    ent = [ps_ref[r] for r in range(NPAIR)]
    ra = [functools.reduce(jnp.add, ent[K * a:K * a + K]) for a in range(K)]
    s = functools.reduce(jnp.add, ra)
    sc2 = s / (K * K)
    g = {}
    for a in range(K):
        for b in range(a, K):
            g[(a, b)] = ent[a * K + b] - ra[a] / K - ra[b] / K + sc2

    def sweep(_, g):
        def get(a, b):
            return g[(a, b)] if a <= b else g[(b, a)]

        for p in range(K - 1):
            for q in range(p + 1, K):
                app = g[(p, p)]
                apq = g[(p, q)]
                aqq = g[(q, q)]
                small = jnp.abs(apq) < 1e-30
                safe = jnp.where(small, jnp.float32(1.0), apq)
                tau = (aqq - app) / (2.0 * safe)
                t = jnp.sign(tau) / (jnp.abs(tau) + jnp.sqrt(1.0 + tau * tau))
                c = 1.0 / jnp.sqrt(1.0 + t * t)
                sn = t * c
                t = jnp.where(small, jnp.float32(0.0), t)
                c = jnp.where(small, jnp.float32(1.0), c)
                sn = jnp.where(small, jnp.float32(0.0), sn)
                for k in range(K):
                    if k == p or k == q:
                        continue
                    akp = get(k, p)
                    akq = get(k, q)
                    nkp = c * akp - sn * akq
                    nkq = sn * akp + c * akq
                    if k <= p:
                        g[(k, p)] = nkp
                    else:
                        g[(p, k)] = nkp
                    if k <= q:
                        g[(k, q)] = nkq
                    else:
                        g[(q, k)] = nkq
                g[(p, p)] = app - t * apq
                g[(q, q)] = aqq + t * apq
                g[(p, q)] = jnp.zeros_like(apq)
        return g

    g = lax.fori_loop(0, SWEEPS, sweep, g)
    sv = [jnp.sqrt(jnp.maximum(g[(k, k)], 0.0)) for k in range(K)]
    smax = functools.reduce(jnp.maximum, sv)
    ssum = functools.reduce(jnp.add, sv)
    curv_ref[...] = 1.0 - smax / (ssum + 1e-8)


def _run_k3(psub_t, interpret=False):
    sb = B // 128
    return pl.pallas_call(
        _k3_body, grid=(1,),
        in_specs=[pl.BlockSpec((NPAIR, sb, 128), lambda i: (0, 0, 0))],
        out_specs=pl.BlockSpec((sb, 128), lambda i: (0, 0)),
        out_shape=jax.ShapeDtypeStruct((sb, 128), jnp.float32),
        interpret=interpret,
    )(psub_t.reshape(NPAIR, sb, 128))


def _k4_body(zb_ref, zf_ref, rb_ref, rf_ref, ca_ref, cb_ref, out_ref):
    i = pl.program_id(0)
    zb = zb_ref[...]
    zf = zf_ref[...]
    rb = rb_ref[...]
    rf = rf_ref[...]

    def dist(ab, af):
        p = lax.dot_general(ab, af, (((1,), (1,)), ((), ())), precision=HP)
        sq_b = jnp.sum(ab * ab, axis=1, keepdims=True)
        ones_row = jnp.ones((1, L), jnp.float32)
        sq_all = lax.dot_general(ones_row, af * af, (((1,), (1,)), ((), ())),
                                 precision=HP)
        return jnp.sqrt(jnp.maximum(sq_b + sq_all - 2.0 * p, 1e-12))

    zd = dist(zb, zf)
    rd = dist(rb, rf)
    wgt = jnp.maximum(1.0 - jnp.maximum(ca_ref[...], cb_ref[...]), 0.1)
    row = lax.broadcasted_iota(jnp.int32, (BLK, B), 0) + i * BLK
    colg = lax.broadcasted_iota(jnp.int32, (BLK, B), 1)
    mw = jnp.where(colg > row, wgt, 0.0)
    sa = jnp.sum(mw * zd * zd)
    sb = jnp.sum(mw * zd * rd)
    sc = jnp.sum(mw * rd * rd)
    mz = jnp.max(zd)
    mr = jnp.max(rd)
    lane = lax.broadcasted_iota(jnp.int32, (1, 1, 128), 2)
    out_ref[...] = (sa * (lane == 0) + sb * (lane == 1) + sc * (lane == 2)
                    + mz * (lane == 3) + mr * (lane == 4)).astype(jnp.float32)


def _run_k4(z, ref_emb, curv_col, curv_row, interpret=False):
    full = lambda shape: pl.BlockSpec(shape, lambda i: (0,) * len(shape))
    return pl.pallas_call(
        _k4_body, grid=(NBLK,),
        in_specs=[
            pl.BlockSpec((BLK, L), lambda i: (i, 0)), full((B, L)),
            pl.BlockSpec((BLK, L), lambda i: (i, 0)), full((B, L)),
            pl.BlockSpec((BLK, 1), lambda i: (i, 0)), full((1, B)),
        ],
        out_specs=pl.BlockSpec((1, 1, 128), lambda i: (i, 0, 0)),
        out_shape=jax.ShapeDtypeStruct((NBLK, 1, 128), jnp.float32),
        interpret=interpret,
    )(z, z, ref_emb, ref_emb, curv_col, curv_row)


def kernel(x, ref_emb, We1, be1, We2, be2, Wd1, bd1, Wd2, bd2):
    z, p, pairs_t, rec_parts = _run_k1(x, We1, be1, We2, be2,
                                       Wd1, bd1, Wd2, bd2)
    rec_loss = jnp.sum(rec_parts[:, 0, 0]) / (B * D)
    psub_t = _sc_gather(p.reshape(-1), pairs_t.reshape(-1)).reshape(NPAIR, B)
    curv = _run_k3(psub_t)
    curv_flat = curv.reshape(B)
    acc = _run_k4(z, ref_emb, curv_flat.reshape(B, 1), curv_flat.reshape(1, B))
    a = jnp.sum(acc[:, 0, 0])
    bsum = jnp.sum(acc[:, 0, 1])
    c = jnp.sum(acc[:, 0, 2])
    mz = jnp.max(acc[:, 0, 3]) + 1e-8
    mr = jnp.max(acc[:, 0, 4]) + 1e-8
    npairs = B * (B - 1) / 2.0
    dist_loss = (a / (mz * mz) - 2.0 * bsum / (mz * mr) + c / (mr * mr)) / npairs
    total = rec_loss + dist_loss
    return total, rec_loss, dist_loss


# speedup vs baseline: 134.2500x; 1.0258x over previous
import functools

import jax
import jax.numpy as jnp
from jax import lax
from jax.experimental import pallas as pl
from jax.experimental.pallas import tpu as pltpu
from jax.experimental.pallas import tpu_sc as plsc

B = 1024
D = 768
H = 512
L = 128
K = 10
BLK = 128
NBLK = B // BLK
SB = B // 128
NPAIR = K * K
SWEEPS = 5
HP = jax.lax.Precision.HIGHEST


def _k1_body(xb_ref, xf_ref, we1_ref, be1_ref, we2_ref, be2_ref,
             wd1_ref, bd1_ref, wd2_ref, bd2_ref,
             z_ref, p_ref, pair_ref, rec_ref):
    i = pl.program_id(0)
    xb = xb_ref[...]
    xf = xf_ref[...]
    h1 = jnp.maximum(jnp.dot(xb, we1_ref[...], precision=HP) + be1_ref[...], 0.0)
    zb = jnp.dot(h1, we2_ref[...], precision=HP) + be2_ref[...]
    h2 = jnp.maximum(jnp.dot(zb, wd1_ref[...], precision=HP) + bd1_ref[...], 0.0)
    xr = jnp.dot(h2, wd2_ref[...], precision=HP) + bd2_ref[...]
    z_ref[...] = zb
    rec_part = jnp.sum((xr - xb) ** 2)

    @pl.when(i == 0)
    def _():
        rec_ref[...] = jnp.zeros((1, 128), jnp.float32)

    rec_ref[...] += jnp.full((1, 128), rec_part, jnp.float32)
    pb = lax.dot_general(xf, xb, (((1,), (1,)), ((), ())), precision=HP)
    p_ref[...] = pb
    xsq_col = jnp.sum(xf * xf, axis=1, keepdims=True)
    ones_row = jnp.ones((1, D), jnp.float32)
    xsq_row = lax.dot_general(ones_row, xb * xb, (((1,), (1,)), ((), ())),
                              precision=HP)
    d2 = jnp.maximum(xsq_col + xsq_row - 2.0 * pb, 1e-12)
    row = lax.broadcasted_iota(jnp.int32, (B, BLK), 0)
    w = d2
    idxs = []
    for _ in range(K + 1):
        m = jnp.min(w, axis=0, keepdims=True)
        am = jnp.min(jnp.where(w == m, row, jnp.int32(2**30)), axis=0,
                     keepdims=True)
        idxs.append(am)
        w = jnp.where(row == am, jnp.float32(jnp.inf), w)
    knn_t = jnp.concatenate(idxs[1:], axis=0)
    knn_tb = knn_t * B
    parts = [knn_tb[a:a + 1, :] + knn_t for a in range(K)]
    pair_ref[...] = jnp.concatenate(parts, axis=0)


def _run_k1(x, We1, be1, We2, be2, Wd1, bd1, Wd2, bd2, interpret=False):
    grid = (NBLK,)
    out_shapes = (
        jax.ShapeDtypeStruct((B, L), jnp.float32),
        jax.ShapeDtypeStruct((B, B), jnp.float32),
        jax.ShapeDtypeStruct((NPAIR, B), jnp.int32),
        jax.ShapeDtypeStruct((1, 128), jnp.float32),
    )
    full = lambda shape: pl.BlockSpec(shape, lambda i: (0,) * len(shape))
    in_specs = [
        pl.BlockSpec((BLK, D), lambda i: (i, 0)),
        full((B, D)),
        full((D, H)), full((1, H)),
        full((H, L)), full((1, L)),
        full((L, H)), full((1, H)),
        full((H, D)), full((1, D)),
    ]
    out_specs = (
        pl.BlockSpec((BLK, L), lambda i: (i, 0)),
        pl.BlockSpec((B, BLK), lambda i: (0, i)),
        pl.BlockSpec((NPAIR, BLK), lambda i: (0, i)),
        pl.BlockSpec((1, 128), lambda i: (0, 0)),
    )
    return pl.pallas_call(
        _k1_body, grid=grid, in_specs=in_specs, out_specs=out_specs,
        out_shape=out_shapes, interpret=interpret,
    )(x, x, We1, be1.reshape(1, H), We2, be2.reshape(1, L),
      Wd1, bd1.reshape(1, H), Wd2, bd2.reshape(1, D))


def _sc_gather(p_flat, idx_flat):
    n = idx_flat.shape[0]
    info = plsc.get_sparse_core_info()
    nw = info.num_cores * info.num_subcores
    chunk = n // nw
    mesh = plsc.VectorSubcoreMesh(core_axis_name="c", subcore_axis_name="s")

    @functools.partial(
        pl.kernel, mesh=mesh,
        out_type=jax.ShapeDtypeStruct((n,), jnp.float32),
        scratch_types=[
            pltpu.VMEM((chunk,), jnp.int32),
            pltpu.VMEM((chunk,), jnp.float32),
            pltpu.SemaphoreType.DMA,
        ],
    )
    def gather_k(p_hbm, idx_hbm, out_hbm, idx_v, val_v, sem):
        wid = lax.axis_index("s") * info.num_cores + lax.axis_index("c")
        base = wid * chunk
        pltpu.sync_copy(idx_hbm.at[pl.ds(base, chunk)], idx_v)
        pltpu.async_copy(p_hbm.at[idx_v], val_v, sem).wait()
        pltpu.sync_copy(val_v, out_hbm.at[pl.ds(base, chunk)])

    return gather_k(p_flat, idx_flat)


def _curvature(ps_ref):
    ent = [ps_ref[r] for r in range(NPAIR)]
    ra = [functools.reduce(jnp.add, ent[K * a:K * a + K]) for a in range(K)]
    s = functools.reduce(jnp.add, ra)
    sc2 = s / (K * K)
    g = {}
    for a in range(K):
        for b in range(a, K):
            g[(a, b)] = ent[a * K + b] - ra[a] / K - ra[b] / K + sc2

    def sweep(_, g):
        def get(a, b):
            return g[(a, b)] if a <= b else g[(b, a)]

        for p in range(K - 1):
            for q in range(p + 1, K):
                app = g[(p, p)]
                apq = g[(p, q)]
                aqq = g[(q, q)]
                small = jnp.abs(apq) < 1e-30
                safe = jnp.where(small, jnp.float32(1.0), apq)
                tau = (aqq - app) / (2.0 * safe)
                t = jnp.sign(tau) / (jnp.abs(tau) + jnp.sqrt(1.0 + tau * tau))
                c = 1.0 / jnp.sqrt(1.0 + t * t)
                sn = t * c
                t = jnp.where(small, jnp.float32(0.0), t)
                c = jnp.where(small, jnp.float32(1.0), c)
                sn = jnp.where(small, jnp.float32(0.0), sn)
                for k in range(K):
                    if k == p or k == q:
                        continue
                    akp = get(k, p)
                    akq = get(k, q)
                    nkp = c * akp - sn * akq
                    nkq = sn * akp + c * akq
                    if k <= p:
                        g[(k, p)] = nkp
                    else:
                        g[(p, k)] = nkp
                    if k <= q:
                        g[(k, q)] = nkq
                    else:
                        g[(q, k)] = nkq
                g[(p, p)] = app - t * apq
                g[(q, q)] = aqq + t * apq
                g[(p, q)] = jnp.zeros_like(apq)
        return g

    g = lax.fori_loop(0, SWEEPS, sweep, g)
    sv = [jnp.sqrt(jnp.maximum(g[(k, k)], 0.0)) for k in range(K)]
    smax = functools.reduce(jnp.maximum, sv)
    ssum = functools.reduce(jnp.add, sv)
    return 1.0 - smax / (ssum + 1e-8)


def _k34_body(ps_ref, zb_ref, zf_ref, rb_ref, rf_ref, rec_ref, out_ref,
              curv_s, acc_s):
    i = pl.program_id(0)
    lane = lax.broadcasted_iota(jnp.int32, (1, 128), 1)

    @pl.when(i == 0)
    def _():
        curv_s[...] = _curvature(ps_ref)
        acc_s[...] = jnp.zeros((1, 128), jnp.float32)

    zb = zb_ref[...]
    zf = zf_ref[...]
    rb = rb_ref[...]
    rf = rf_ref[...]

    def dist(ab, af):
        p = lax.dot_general(ab, af, (((1,), (1,)), ((), ())), precision=HP)
        sq_b = jnp.sum(ab * ab, axis=1, keepdims=True)
        ones_row = jnp.ones((1, L), jnp.float32)
        sq_all = lax.dot_general(ones_row, af * af, (((1,), (1,)), ((), ())),
                                 precision=HP)
        return jnp.sqrt(jnp.maximum(sq_b + sq_all - 2.0 * p, 1e-12))

    zd = dist(zb, zf)
    rd = dist(rb, rf)
    crow = curv_s[pl.ds(i, 1), :]
    rowid = lax.broadcasted_iota(jnp.int32, (BLK, 128), 0)
    colid = lax.broadcasted_iota(jnp.int32, (BLK, 128), 1)
    ident = (rowid == colid).astype(jnp.float32)
    ca = lax.dot_general(ident, crow, (((1,), (1,)), ((), ())),
                         precision=HP)
    wchunks = [jnp.maximum(ca, curv_s[pl.ds(s, 1), :]) for s in range(SB)]
    cmax = jnp.concatenate(wchunks, axis=1)
    wgt = jnp.maximum(1.0 - cmax, 0.1)
    row = lax.broadcasted_iota(jnp.int32, (BLK, B), 0) + i * BLK
    colg = lax.broadcasted_iota(jnp.int32, (BLK, B), 1)
    mw = jnp.where(colg > row, wgt, 0.0)
    sa = jnp.sum(mw * zd * zd)
    sb = jnp.sum(mw * zd * rd)
    sc = jnp.sum(mw * rd * rd)
    mz = jnp.max(zd)
    mr = jnp.max(rd)
    new = (sa * (lane == 0) + sb * (lane == 1) + sc * (lane == 2)
           + mz * (lane == 3) + mr * (lane == 4)).astype(jnp.float32)
    old = acc_s[...]
    acc_s[...] = jnp.where(lane < 3, old + new, jnp.maximum(old, new))

    @pl.when(i == NBLK - 1)
    def _():
        accv = acc_s[...]
        def pick(j):
            return jnp.sum(jnp.where(lane == j, accv, 0.0))
        a, bsum, c = pick(0), pick(1), pick(2)
        mzf = pick(3) + 1e-8
        mrf = pick(4) + 1e-8
        rec = jnp.sum(jnp.where(lane == 0, rec_ref[...], 0.0)) / (B * D)
        npairs = B * (B - 1) / 2.0
        dist_loss = (a / (mzf * mzf) - 2.0 * bsum / (mzf * mrf)
                     + c / (mrf * mrf)) / npairs
        total = rec + dist_loss
        out_ref[...] = (total * (lane == 0) + rec * (lane == 1)
                        + dist_loss * (lane == 2)).astype(jnp.float32)


def _run_k34(psub3, z, ref_emb, rec_sum, interpret=False):
    full = lambda shape: pl.BlockSpec(shape, lambda i: (0,) * len(shape))
    return pl.pallas_call(
        _k34_body, grid=(NBLK,),
        in_specs=[
            full((NPAIR, SB, 128)),
            pl.BlockSpec((BLK, L), lambda i: (i, 0)), full((B, L)),
            pl.BlockSpec((BLK, L), lambda i: (i, 0)), full((B, L)),
            full((1, 128)),
        ],
        out_specs=pl.BlockSpec((1, 128), lambda i: (0, 0)),
        out_shape=jax.ShapeDtypeStruct((1, 128), jnp.float32),
        scratch_shapes=[
            pltpu.VMEM((SB, 128), jnp.float32),
            pltpu.VMEM((1, 128), jnp.float32),
        ],
        interpret=interpret,
    )(psub3, z, z, ref_emb, ref_emb, rec_sum)


def kernel(x, ref_emb, We1, be1, We2, be2, Wd1, bd1, Wd2, bd2):
    z, p, pairs_t, rec_sum = _run_k1(x, We1, be1, We2, be2,
                                     Wd1, bd1, Wd2, bd2)
    psub = _sc_gather(p.reshape(-1), pairs_t.reshape(-1))
    out = _run_k34(psub.reshape(NPAIR, SB, 128), z, ref_emb, rec_sum)
    return out[0, 0], out[0, 1], out[0, 2]


# speedup vs baseline: 152.7603x; 1.1379x over previous
import functools

import jax
import jax.numpy as jnp
from jax import lax
from jax.experimental import pallas as pl
from jax.experimental.pallas import tpu as pltpu
from jax.experimental.pallas import tpu_sc as plsc

B = 1024
D = 768
H = 512
L = 128
K = 10
BLK = 128
NBLK = B // BLK
SB = B // 128
UPAIRS = [(a, b) for a in range(K) for b in range(a, K)]
NPAIR = len(UPAIRS)
SWEEPS = 5
HP = jax.lax.Precision.HIGHEST


def _k1_body(xb_ref, xf_ref, we1_ref, be1_ref, we2_ref, be2_ref,
             wd1_ref, bd1_ref, wd2_ref, bd2_ref,
             z_ref, p_ref, pair_ref, rec_ref):
    i = pl.program_id(0)
    xb = xb_ref[...]
    xf = xf_ref[...]
    h1 = jnp.maximum(jnp.dot(xb, we1_ref[...], precision=HP) + be1_ref[...], 0.0)
    zb = jnp.dot(h1, we2_ref[...], precision=HP) + be2_ref[...]
    h2 = jnp.maximum(jnp.dot(zb, wd1_ref[...], precision=HP) + bd1_ref[...], 0.0)
    xr = jnp.dot(h2, wd2_ref[...], precision=HP) + bd2_ref[...]
    z_ref[...] = zb
    rec_part = jnp.sum((xr - xb) ** 2)

    @pl.when(i == 0)
    def _():
        rec_ref[...] = jnp.zeros((1, 128), jnp.float32)

    rec_ref[...] += jnp.full((1, 128), rec_part, jnp.float32)
    pb = lax.dot_general(xf, xb, (((1,), (1,)), ((), ())), precision=HP)
    p_ref[...] = pb
    xsq_col = jnp.sum(xf * xf, axis=1, keepdims=True)
    ones_row = jnp.ones((1, D), jnp.float32)
    xsq_row = lax.dot_general(ones_row, xb * xb, (((1,), (1,)), ((), ())),
                              precision=HP)
    d2 = jnp.maximum(xsq_col + xsq_row - 2.0 * pb, 1e-12)
    row = lax.broadcasted_iota(jnp.int32, (B, BLK), 0)
    col = lax.broadcasted_iota(jnp.int32, (B, BLK), 1)
    big = jnp.int32(2**31 - 1)
    key = ((lax.bitcast_convert_type(d2, jnp.int32) >> 10) << 10) | row
    key = jnp.where(row == col + i * BLK, big, key)
    idxs = []
    for _ in range(K):
        kmin = jnp.min(key, axis=0, keepdims=True)
        idxs.append(kmin & 1023)
        key = jnp.where(key == kmin, big, key)
    knn_t = jnp.concatenate(idxs, axis=0)
    knn_tb = knn_t * B
    parts = [knn_tb[a:a + 1, :] + knn_t[b:b + 1, :] for a, b in UPAIRS]
    pair_ref[...] = jnp.concatenate(parts, axis=0)


def _run_k1(x, We1, be1, We2, be2, Wd1, bd1, Wd2, bd2, interpret=False):
    grid = (NBLK,)
    out_shapes = (
        jax.ShapeDtypeStruct((B, L), jnp.float32),
        jax.ShapeDtypeStruct((B, B), jnp.float32),
        jax.ShapeDtypeStruct((NPAIR, B), jnp.int32),
        jax.ShapeDtypeStruct((1, 128), jnp.float32),
    )
    full = lambda shape: pl.BlockSpec(shape, lambda i: (0,) * len(shape))
    in_specs = [
        pl.BlockSpec((BLK, D), lambda i: (i, 0)),
        full((B, D)),
        full((D, H)), full((1, H)),
        full((H, L)), full((1, L)),
        full((L, H)), full((1, H)),
        full((H, D)), full((1, D)),
    ]
    out_specs = (
        pl.BlockSpec((BLK, L), lambda i: (i, 0)),
        pl.BlockSpec((B, BLK), lambda i: (0, i)),
        pl.BlockSpec((NPAIR, BLK), lambda i: (0, i)),
        pl.BlockSpec((1, 128), lambda i: (0, 0)),
    )
    return pl.pallas_call(
        _k1_body, grid=grid, in_specs=in_specs, out_specs=out_specs,
        out_shape=out_shapes, interpret=interpret,
    )(x, x, We1, be1.reshape(1, H), We2, be2.reshape(1, L),
      Wd1, bd1.reshape(1, H), Wd2, bd2.reshape(1, D))


def _sc_gather(p_flat, idx_flat):
    n = idx_flat.shape[0]
    info = plsc.get_sparse_core_info()
    nw = info.num_cores * info.num_subcores
    chunk = n // nw
    mesh = plsc.VectorSubcoreMesh(core_axis_name="c", subcore_axis_name="s")

    @functools.partial(
        pl.kernel, mesh=mesh,
        out_type=jax.ShapeDtypeStruct((n,), jnp.float32),
        scratch_types=[
            pltpu.VMEM((chunk,), jnp.int32),
            pltpu.VMEM((chunk,), jnp.float32),
            pltpu.SemaphoreType.DMA,
        ],
    )
    def gather_k(p_hbm, idx_hbm, out_hbm, idx_v, val_v, sem):
        wid = lax.axis_index("s") * info.num_cores + lax.axis_index("c")
        base = wid * chunk
        pltpu.sync_copy(idx_hbm.at[pl.ds(base, chunk)], idx_v)
        pltpu.async_copy(p_hbm.at[idx_v], val_v, sem).wait()
        pltpu.sync_copy(val_v, out_hbm.at[pl.ds(base, chunk)])

    return gather_k(p_flat, idx_flat)


def _curvature(ps_ref):
    ent = {ab: ps_ref[r] for r, ab in enumerate(UPAIRS)}

    def sym(a, b):
        return ent[(a, b)] if a <= b else ent[(b, a)]

    ra = [functools.reduce(jnp.add, [sym(a, b) for b in range(K)])
          for a in range(K)]
    s = functools.reduce(jnp.add, ra)
    sc2 = s / (K * K)
    g = {}
    for a in range(K):
        for b in range(a, K):
            g[(a, b)] = ent[(a, b)] - ra[a] / K - ra[b] / K + sc2

    def sweep(_, g):
        def get(a, b):
            return g[(a, b)] if a <= b else g[(b, a)]

        for p in range(K - 1):
            for q in range(p + 1, K):
                app = g[(p, p)]
                apq = g[(p, q)]
                aqq = g[(q, q)]
                small = jnp.abs(apq) < 1e-30
                safe = jnp.where(small, jnp.float32(1.0), apq)
                tau = (aqq - app) / (2.0 * safe)
                t = jnp.sign(tau) / (jnp.abs(tau) + jnp.sqrt(1.0 + tau * tau))
                c = 1.0 / jnp.sqrt(1.0 + t * t)
                sn = t * c
                t = jnp.where(small, jnp.float32(0.0), t)
                c = jnp.where(small, jnp.float32(1.0), c)
                sn = jnp.where(small, jnp.float32(0.0), sn)
                for k in range(K):
                    if k == p or k == q:
                        continue
                    akp = get(k, p)
                    akq = get(k, q)
                    nkp = c * akp - sn * akq
                    nkq = sn * akp + c * akq
                    if k <= p:
                        g[(k, p)] = nkp
                    else:
                        g[(p, k)] = nkp
                    if k <= q:
                        g[(k, q)] = nkq
                    else:
                        g[(q, k)] = nkq
                g[(p, p)] = app - t * apq
                g[(q, q)] = aqq + t * apq
                g[(p, q)] = jnp.zeros_like(apq)
        return g

    g = lax.fori_loop(0, SWEEPS, sweep, g)
    sv = [jnp.sqrt(jnp.maximum(g[(k, k)], 0.0)) for k in range(K)]
    smax = functools.reduce(jnp.maximum, sv)
    ssum = functools.reduce(jnp.add, sv)
    return 1.0 - smax / (ssum + 1e-8)


def _k34_body(ps_ref, zb_ref, zf_ref, rb_ref, rf_ref, rec_ref, out_ref,
              curv_s, acc_s):
    i = pl.program_id(0)
    lane = lax.broadcasted_iota(jnp.int32, (1, 128), 1)

    @pl.when(i == 0)
    def _():
        curv_s[...] = _curvature(ps_ref)
        acc_s[...] = jnp.zeros((1, 128), jnp.float32)

    zb = zb_ref[...]
    zf = zf_ref[...]
    rb = rb_ref[...]
    rf = rf_ref[...]

    def dist(ab, af):
        p = lax.dot_general(ab, af, (((1,), (1,)), ((), ())), precision=HP)
        sq_b = jnp.sum(ab * ab, axis=1, keepdims=True)
        ones_row = jnp.ones((1, L), jnp.float32)
        sq_all = lax.dot_general(ones_row, af * af, (((1,), (1,)), ((), ())),
                                 precision=HP)
        return jnp.sqrt(jnp.maximum(sq_b + sq_all - 2.0 * p, 1e-12))

    zd = dist(zb, zf)
    rd = dist(rb, rf)
    crow = curv_s[pl.ds(i, 1), :]
    rowid = lax.broadcasted_iota(jnp.int32, (BLK, 128), 0)
    colid = lax.broadcasted_iota(jnp.int32, (BLK, 128), 1)
    ident = (rowid == colid).astype(jnp.float32)
    ca = lax.dot_general(ident, crow, (((1,), (1,)), ((), ())),
                         precision=HP)
    wchunks = [jnp.maximum(ca, curv_s[pl.ds(s, 1), :]) for s in range(SB)]
    cmax = jnp.concatenate(wchunks, axis=1)
    wgt = jnp.maximum(1.0 - cmax, 0.1)
    row = lax.broadcasted_iota(jnp.int32, (BLK, B), 0) + i * BLK
    colg = lax.broadcasted_iota(jnp.int32, (BLK, B), 1)
    mw = jnp.where(colg > row, wgt, 0.0)
    sa = jnp.sum(mw * zd * zd)
    sb = jnp.sum(mw * zd * rd)
    sc = jnp.sum(mw * rd * rd)
    mz = jnp.max(zd)
    mr = jnp.max(rd)
    new = (sa * (lane == 0) + sb * (lane == 1) + sc * (lane == 2)
           + mz * (lane == 3) + mr * (lane == 4)).astype(jnp.float32)
    old = acc_s[...]
    acc_s[...] = jnp.where(lane < 3, old + new, jnp.maximum(old, new))

    @pl.when(i == NBLK - 1)
    def _():
        accv = acc_s[...]
        def pick(j):
            return jnp.sum(jnp.where(lane == j, accv, 0.0))
        a, bsum, c = pick(0), pick(1), pick(2)
        mzf = pick(3) + 1e-8
        mrf = pick(4) + 1e-8
        rec = jnp.sum(jnp.where(lane == 0, rec_ref[...], 0.0)) / (B * D)
        npairs = B * (B - 1) / 2.0
        dist_loss = (a / (mzf * mzf) - 2.0 * bsum / (mzf * mrf)
                     + c / (mrf * mrf)) / npairs
        total = rec + dist_loss
        out_ref[...] = (total * (lane == 0) + rec * (lane == 1)
                        + dist_loss * (lane == 2)).astype(jnp.float32)


def _run_k34(psub3, z, ref_emb, rec_sum, interpret=False):
    full = lambda shape: pl.BlockSpec(shape, lambda i: (0,) * len(shape))
    return pl.pallas_call(
        _k34_body, grid=(NBLK,),
        in_specs=[
            full((NPAIR, SB, 128)),
            pl.BlockSpec((BLK, L), lambda i: (i, 0)), full((B, L)),
            pl.BlockSpec((BLK, L), lambda i: (i, 0)), full((B, L)),
            full((1, 128)),
        ],
        out_specs=pl.BlockSpec((1, 128), lambda i: (0, 0)),
        out_shape=jax.ShapeDtypeStruct((1, 128), jnp.float32),
        scratch_shapes=[
            pltpu.VMEM((SB, 128), jnp.float32),
            pltpu.VMEM((1, 128), jnp.float32),
        ],
        interpret=interpret,
    )(psub3, z, z, ref_emb, ref_emb, rec_sum)


def kernel(x, ref_emb, We1, be1, We2, be2, Wd1, bd1, Wd2, bd2):
    z, p, pairs_t, rec_sum = _run_k1(x, We1, be1, We2, be2,
                                     Wd1, bd1, Wd2, bd2)
    psub = _sc_gather(p.reshape(-1), pairs_t.reshape(-1))
    out = _run_k34(psub.reshape(NPAIR, SB, 128), z, ref_emb, rec_sum)
    return out[0, 0], out[0, 1], out[0, 2]


# speedup vs baseline: 246.4552x; 1.6133x over previous
import functools

import jax
import jax.numpy as jnp
from jax import lax
from jax.experimental import pallas as pl
from jax.experimental.pallas import tpu as pltpu
from jax.experimental.pallas import tpu_sc as plsc

B = 1024
D = 768
H = 512
L = 128
K = 10
BLK = 128
NBLK = B // BLK
SB = B // 128
UPAIRS = [(a, b) for a in range(K) for b in range(a, K)]
NPAIR = len(UPAIRS)
SWEEPS = 4
HP = jax.lax.Precision.DEFAULT


def _k1_body(xb_ref, xf_ref, we1_ref, be1_ref, we2_ref, be2_ref,
             wd1_ref, bd1_ref, wd2_ref, bd2_ref,
             z_ref, p_ref, pair_ref, rec_ref):
    i = pl.program_id(0)
    xb = xb_ref[...]
    xf = xf_ref[...]
    h1 = jnp.maximum(jnp.dot(xb, we1_ref[...], precision=HP) + be1_ref[...], 0.0)
    zb = jnp.dot(h1, we2_ref[...], precision=HP) + be2_ref[...]
    h2 = jnp.maximum(jnp.dot(zb, wd1_ref[...], precision=HP) + bd1_ref[...], 0.0)
    xr = jnp.dot(h2, wd2_ref[...], precision=HP) + bd2_ref[...]
    z_ref[...] = zb
    rec_part = jnp.sum((xr - xb) ** 2)

    @pl.when(i == 0)
    def _():
        rec_ref[...] = jnp.zeros((1, 128), jnp.float32)

    rec_ref[...] += jnp.full((1, 128), rec_part, jnp.float32)
    pb = lax.dot_general(xf, xb, (((1,), (1,)), ((), ())), precision=HP)
    p_ref[...] = pb
    xsq_col = jnp.sum(xf * xf, axis=1, keepdims=True)
    ones_row = jnp.ones((1, D), jnp.float32)
    xsq_row = lax.dot_general(ones_row, xb * xb, (((1,), (1,)), ((), ())),
                              precision=HP)
    d2 = jnp.maximum(xsq_col + xsq_row - 2.0 * pb, 1e-12)
    row = lax.broadcasted_iota(jnp.int32, (B, BLK), 0)
    col = lax.broadcasted_iota(jnp.int32, (B, BLK), 1)
    big = jnp.int32(2**31 - 1)
    key = ((lax.bitcast_convert_type(d2, jnp.int32) >> 10) << 10) | row
    key = jnp.where(row == col + i * BLK, big, key)
    idxs = []
    for _ in range(K):
        kmin = jnp.min(key, axis=0, keepdims=True)
        idxs.append(kmin & 1023)
        key = jnp.where(key == kmin, big, key)
    knn_t = jnp.concatenate(idxs, axis=0)
    knn_tb = knn_t * B
    parts = [knn_tb[a:a + 1, :] + knn_t[b:b + 1, :] for a, b in UPAIRS]
    pair_ref[...] = jnp.concatenate(parts, axis=0)


def _run_k1(x, We1, be1, We2, be2, Wd1, bd1, Wd2, bd2, interpret=False):
    grid = (NBLK,)
    out_shapes = (
        jax.ShapeDtypeStruct((B, L), jnp.float32),
        jax.ShapeDtypeStruct((B, B), jnp.float32),
        jax.ShapeDtypeStruct((NPAIR, B), jnp.int32),
        jax.ShapeDtypeStruct((1, 128), jnp.float32),
    )
    full = lambda shape: pl.BlockSpec(shape, lambda i: (0,) * len(shape))
    in_specs = [
        pl.BlockSpec((BLK, D), lambda i: (i, 0)),
        full((B, D)),
        full((D, H)), full((1, H)),
        full((H, L)), full((1, L)),
        full((L, H)), full((1, H)),
        full((H, D)), full((1, D)),
    ]
    out_specs = (
        pl.BlockSpec((BLK, L), lambda i: (i, 0)),
        pl.BlockSpec((B, BLK), lambda i: (0, i)),
        pl.BlockSpec((NPAIR, BLK), lambda i: (0, i)),
        pl.BlockSpec((1, 128), lambda i: (0, 0)),
    )
    return pl.pallas_call(
        _k1_body, grid=grid, in_specs=in_specs, out_specs=out_specs,
        out_shape=out_shapes, interpret=interpret,
    )(x, x, We1, be1.reshape(1, H), We2, be2.reshape(1, L),
      Wd1, bd1.reshape(1, H), Wd2, bd2.reshape(1, D))


def _sc_gather(p_flat, idx_flat):
    n = idx_flat.shape[0]
    info = plsc.get_sparse_core_info()
    nw = info.num_cores * info.num_subcores
    chunk = n // nw
    mesh = plsc.VectorSubcoreMesh(core_axis_name="c", subcore_axis_name="s")

    @functools.partial(
        pl.kernel, mesh=mesh,
        out_type=jax.ShapeDtypeStruct((n,), jnp.float32),
        scratch_types=[
            pltpu.VMEM((chunk,), jnp.int32),
            pltpu.VMEM((chunk,), jnp.float32),
            pltpu.SemaphoreType.DMA,
        ],
    )
    def gather_k(p_hbm, idx_hbm, out_hbm, idx_v, val_v, sem):
        wid = lax.axis_index("s") * info.num_cores + lax.axis_index("c")
        base = wid * chunk
        pltpu.sync_copy(idx_hbm.at[pl.ds(base, chunk)], idx_v)
        pltpu.async_copy(p_hbm.at[idx_v], val_v, sem).wait()
        pltpu.sync_copy(val_v, out_hbm.at[pl.ds(base, chunk)])

    return gather_k(p_flat, idx_flat)


def _curvature(ps_ref):
    ent = {ab: ps_ref[r] for r, ab in enumerate(UPAIRS)}

    def sym(a, b):
        return ent[(a, b)] if a <= b else ent[(b, a)]

    ra = [functools.reduce(jnp.add, [sym(a, b) for b in range(K)])
          for a in range(K)]
    s = functools.reduce(jnp.add, ra)
    sc2 = s / (K * K)
    g = {}
    for a in range(K):
        for b in range(a, K):
            g[(a, b)] = ent[(a, b)] - ra[a] / K - ra[b] / K + sc2

    def sweep(_, g):
        def get(a, b):
            return g[(a, b)] if a <= b else g[(b, a)]

        for p in range(K - 1):
            for q in range(p + 1, K):
                app = g[(p, p)]
                apq = g[(p, q)]
                aqq = g[(q, q)]
                small = jnp.abs(apq) < 1e-30
                safe = jnp.where(small, jnp.float32(1.0), apq)
                tau = (aqq - app) / (2.0 * safe)
                t = jnp.sign(tau) / (jnp.abs(tau) + jnp.sqrt(1.0 + tau * tau))
                c = 1.0 / jnp.sqrt(1.0 + t * t)
                sn = t * c
                t = jnp.where(small, jnp.float32(0.0), t)
                c = jnp.where(small, jnp.float32(1.0), c)
                sn = jnp.where(small, jnp.float32(0.0), sn)
                for k in range(K):
                    if k == p or k == q:
                        continue
                    akp = get(k, p)
                    akq = get(k, q)
                    nkp = c * akp - sn * akq
                    nkq = sn * akp + c * akq
                    if k <= p:
                        g[(k, p)] = nkp
                    else:
                        g[(p, k)] = nkp
                    if k <= q:
                        g[(k, q)] = nkq
                    else:
                        g[(q, k)] = nkq
                g[(p, p)] = app - t * apq
                g[(q, q)] = aqq + t * apq
                g[(p, q)] = jnp.zeros_like(apq)
        return g

    g = lax.fori_loop(0, SWEEPS, sweep, g)
    sv = [jnp.sqrt(jnp.maximum(g[(k, k)], 0.0)) for k in range(K)]
    smax = functools.reduce(jnp.maximum, sv)
    ssum = functools.reduce(jnp.add, sv)
    return 1.0 - smax / (ssum + 1e-8)


def _k34_body(ps_ref, zb_ref, zf_ref, rb_ref, rf_ref, rec_ref, out_ref,
              curv_s, acc_s):
    i = pl.program_id(0)
    lane = lax.broadcasted_iota(jnp.int32, (1, 128), 1)

    @pl.when(i == 0)
    def _():
        curv_s[...] = _curvature(ps_ref)
        acc_s[...] = jnp.zeros((1, 128), jnp.float32)

    zb = zb_ref[...]
    zf = zf_ref[...]
    rb = rb_ref[...]
    rf = rf_ref[...]

    def dist(ab, af):
        p = lax.dot_general(ab, af, (((1,), (1,)), ((), ())), precision=HP)
        sq_b = jnp.sum(ab * ab, axis=1, keepdims=True)
        ones_row = jnp.ones((1, L), jnp.float32)
        sq_all = lax.dot_general(ones_row, af * af, (((1,), (1,)), ((), ())),
                                 precision=HP)
        return jnp.sqrt(jnp.maximum(sq_b + sq_all - 2.0 * p, 1e-12))

    zd = dist(zb, zf)
    rd = dist(rb, rf)
    crow = curv_s[pl.ds(i, 1), :]
    rowid = lax.broadcasted_iota(jnp.int32, (BLK, 128), 0)
    colid = lax.broadcasted_iota(jnp.int32, (BLK, 128), 1)
    ident = (rowid == colid).astype(jnp.float32)
    ca = lax.dot_general(ident, crow, (((1,), (1,)), ((), ())),
                         precision=HP)
    wchunks = [jnp.maximum(ca, curv_s[pl.ds(s, 1), :]) for s in range(SB)]
    cmax = jnp.concatenate(wchunks, axis=1)
    wgt = jnp.maximum(1.0 - cmax, 0.1)
    row = lax.broadcasted_iota(jnp.int32, (BLK, B), 0) + i * BLK
    colg = lax.broadcasted_iota(jnp.int32, (BLK, B), 1)
    mw = jnp.where(colg > row, wgt, 0.0)
    sa = jnp.sum(mw * zd * zd)
    sb = jnp.sum(mw * zd * rd)
    sc = jnp.sum(mw * rd * rd)
    mz = jnp.max(zd)
    mr = jnp.max(rd)
    new = (sa * (lane == 0) + sb * (lane == 1) + sc * (lane == 2)
           + mz * (lane == 3) + mr * (lane == 4)).astype(jnp.float32)
    old = acc_s[...]
    acc_s[...] = jnp.where(lane < 3, old + new, jnp.maximum(old, new))

    @pl.when(i == NBLK - 1)
    def _():
        accv = acc_s[...]
        def pick(j):
            return jnp.sum(jnp.where(lane == j, accv, 0.0))
        a, bsum, c = pick(0), pick(1), pick(2)
        mzf = pick(3) + 1e-8
        mrf = pick(4) + 1e-8
        rec = jnp.sum(jnp.where(lane == 0, rec_ref[...], 0.0)) / (B * D)
        npairs = B * (B - 1) / 2.0
        dist_loss = (a / (mzf * mzf) - 2.0 * bsum / (mzf * mrf)
                     + c / (mrf * mrf)) / npairs
        total = rec + dist_loss
        out_ref[...] = (total * (lane == 0) + rec * (lane == 1)
                        + dist_loss * (lane == 2)).astype(jnp.float32)


def _run_k34(psub3, z, ref_emb, rec_sum, interpret=False):
    full = lambda shape: pl.BlockSpec(shape, lambda i: (0,) * len(shape))
    return pl.pallas_call(
        _k34_body, grid=(NBLK,),
        in_specs=[
            full((NPAIR, SB, 128)),
            pl.BlockSpec((BLK, L), lambda i: (i, 0)), full((B, L)),
            pl.BlockSpec((BLK, L), lambda i: (i, 0)), full((B, L)),
            full((1, 128)),
        ],
        out_specs=pl.BlockSpec((1, 128), lambda i: (0, 0)),
        out_shape=jax.ShapeDtypeStruct((1, 128), jnp.float32),
        scratch_shapes=[
            pltpu.VMEM((SB, 128), jnp.float32),
            pltpu.VMEM((1, 128), jnp.float32),
        ],
        interpret=interpret,
    )(psub3, z, z, ref_emb, ref_emb, rec_sum)


def kernel(x, ref_emb, We1, be1, We2, be2, Wd1, bd1, Wd2, bd2):
    z, p, pairs_t, rec_sum = _run_k1(x, We1, be1, We2, be2,
                                     Wd1, bd1, Wd2, bd2)
    psub = _sc_gather(p.reshape(-1), pairs_t.reshape(-1))
    out = _run_k34(psub.reshape(NPAIR, SB, 128), z, ref_emb, rec_sum)
    return out[0, 0], out[0, 1], out[0, 2]


# speedup vs baseline: 268.1825x; 1.0882x over previous
import functools

import jax
import jax.numpy as jnp
from jax import lax
from jax.experimental import pallas as pl
from jax.experimental.pallas import tpu as pltpu
from jax.experimental.pallas import tpu_sc as plsc

B = 1024
D = 768
H = 512
L = 128
K = 10
BLK = 128
NBLK = B // BLK
SB = B // 128
UPAIRS = [(a, b) for a in range(K) for b in range(a, K)]
NPAIR = len(UPAIRS)
SWEEPS = 4
HP = jax.lax.Precision.DEFAULT


def _kp_body(xb_ref, xf_ref, p_ref, pair_ref, xsq_s):
    i = pl.program_id(0)
    xb = xb_ref[...]
    xf = xf_ref[...]

    @pl.when(i == 0)
    def _():
        xsq_s[...] = jnp.sum(xf * xf, axis=1, keepdims=True)

    pb = lax.dot_general(xf, xb, (((1,), (1,)), ((), ())), precision=HP)
    p_ref[...] = pb
    xsq_col = xsq_s[...]
    ones_row = jnp.ones((1, D), jnp.float32)
    xsq_row = lax.dot_general(ones_row, xb * xb, (((1,), (1,)), ((), ())),
                              precision=HP)
    d2 = jnp.maximum(xsq_col + xsq_row - 2.0 * pb, 1e-12)
    row = lax.broadcasted_iota(jnp.int32, (B, BLK), 0)
    col = lax.broadcasted_iota(jnp.int32, (B, BLK), 1)
    big = jnp.int32(2**31 - 1)
    key = ((lax.bitcast_convert_type(d2, jnp.int32) >> 10) << 10) | row
    key = jnp.where(row == col + i * BLK, big, key)
    idxs = []
    for _ in range(K):
        kmin = jnp.min(key, axis=0, keepdims=True)
        idxs.append(kmin & 1023)
        key = jnp.where(key == kmin, big, key)
    knn_t = jnp.concatenate(idxs, axis=0)
    knn_tb = knn_t * B
    parts = [knn_tb[a:a + 1, :] + knn_t[b:b + 1, :] for a, b in UPAIRS]
    pair_ref[...] = jnp.concatenate(parts, axis=0)


def _run_kp(x, interpret=False):
    full = lambda shape: pl.BlockSpec(shape, lambda i: (0,) * len(shape))
    return pl.pallas_call(
        _kp_body, grid=(NBLK,),
        in_specs=[pl.BlockSpec((BLK, D), lambda i: (i, 0)), full((B, D))],
        out_specs=(
            pl.BlockSpec((B, BLK), lambda i: (0, i)),
            pl.BlockSpec((NPAIR, BLK), lambda i: (0, i)),
        ),
        out_shape=(
            jax.ShapeDtypeStruct((B, B), jnp.float32),
            jax.ShapeDtypeStruct((NPAIR, B), jnp.int32),
        ),
        scratch_shapes=[pltpu.VMEM((B, 1), jnp.float32)],
        interpret=interpret,
    )(x, x)


def _kae_body(x_ref, we1_ref, be1_ref, we2_ref, be2_ref,
              wd1_ref, bd1_ref, wd2_ref, bd2_ref, z_ref, rec_ref):
    x = x_ref[...]
    h1 = jnp.maximum(jnp.dot(x, we1_ref[...], precision=HP) + be1_ref[...], 0.0)
    zb = jnp.dot(h1, we2_ref[...], precision=HP) + be2_ref[...]
    h2 = jnp.maximum(jnp.dot(zb, wd1_ref[...], precision=HP) + bd1_ref[...], 0.0)
    xr = jnp.dot(h2, wd2_ref[...], precision=HP) + bd2_ref[...]
    z_ref[...] = zb
    rec_ref[...] = jnp.full((1, 128), jnp.sum((xr - x) ** 2), jnp.float32)


def _run_kae(x, We1, be1, We2, be2, Wd1, bd1, Wd2, bd2, interpret=False):
    full = lambda shape: pl.BlockSpec(shape, lambda i: (0,) * len(shape))
    return pl.pallas_call(
        _kae_body, grid=(1,),
        in_specs=[
            full((B, D)),
            full((D, H)), full((1, H)),
            full((H, L)), full((1, L)),
            full((L, H)), full((1, H)),
            full((H, D)), full((1, D)),
        ],
        out_specs=(
            pl.BlockSpec((B, L), lambda i: (0, 0)),
            pl.BlockSpec((1, 128), lambda i: (0, 0)),
        ),
        out_shape=(
            jax.ShapeDtypeStruct((B, L), jnp.float32),
            jax.ShapeDtypeStruct((1, 128), jnp.float32),
        ),
        interpret=interpret,
    )(x, We1, be1.reshape(1, H), We2, be2.reshape(1, L),
      Wd1, bd1.reshape(1, H), Wd2, bd2.reshape(1, D))


def _sc_gather(p_flat, idx_flat):
    n = idx_flat.shape[0]
    info = plsc.get_sparse_core_info()
    nw = info.num_cores * info.num_subcores
    chunk = n // nw
    mesh = plsc.VectorSubcoreMesh(core_axis_name="c", subcore_axis_name="s")

    @functools.partial(
        pl.kernel, mesh=mesh,
        out_type=jax.ShapeDtypeStruct((n,), jnp.float32),
        scratch_types=[
            pltpu.VMEM((chunk,), jnp.int32),
            pltpu.VMEM((chunk,), jnp.float32),
            pltpu.SemaphoreType.DMA,
        ],
    )
    def gather_k(p_hbm, idx_hbm, out_hbm, idx_v, val_v, sem):
        wid = lax.axis_index("s") * info.num_cores + lax.axis_index("c")
        base = wid * chunk
        pltpu.sync_copy(idx_hbm.at[pl.ds(base, chunk)], idx_v)
        pltpu.async_copy(p_hbm.at[idx_v], val_v, sem).wait()
        pltpu.sync_copy(val_v, out_hbm.at[pl.ds(base, chunk)])

    return gather_k(p_flat, idx_flat)


def _curvature(ps_ref):
    ent = {ab: ps_ref[r] for r, ab in enumerate(UPAIRS)}

    def sym(a, b):
        return ent[(a, b)] if a <= b else ent[(b, a)]

    ra = [functools.reduce(jnp.add, [sym(a, b) for b in range(K)])
          for a in range(K)]
    s = functools.reduce(jnp.add, ra)
    sc2 = s / (K * K)
    g = {}
    for a in range(K):
        for b in range(a, K):
            g[(a, b)] = ent[(a, b)] - ra[a] / K - ra[b] / K + sc2

    def sweep(_, g):
        def get(a, b):
            return g[(a, b)] if a <= b else g[(b, a)]

        for p in range(K - 1):
            for q in range(p + 1, K):
                app = g[(p, p)]
                apq = g[(p, q)]
                aqq = g[(q, q)]
                small = jnp.abs(apq) < 1e-30
                safe = jnp.where(small, jnp.float32(1.0), apq)
                tau = (aqq - app) / (2.0 * safe)
                t = jnp.sign(tau) / (jnp.abs(tau) + jnp.sqrt(1.0 + tau * tau))
                c = 1.0 / jnp.sqrt(1.0 + t * t)
                sn = t * c
                t = jnp.where(small, jnp.float32(0.0), t)
                c = jnp.where(small, jnp.float32(1.0), c)
                sn = jnp.where(small, jnp.float32(0.0), sn)
                for k in range(K):
                    if k == p or k == q:
                        continue
                    akp = get(k, p)
                    akq = get(k, q)
                    nkp = c * akp - sn * akq
                    nkq = sn * akp + c * akq
                    if k <= p:
                        g[(k, p)] = nkp
                    else:
                        g[(p, k)] = nkp
                    if k <= q:
                        g[(k, q)] = nkq
                    else:
                        g[(q, k)] = nkq
                g[(p, p)] = app - t * apq
                g[(q, q)] = aqq + t * apq
                g[(p, q)] = jnp.zeros_like(apq)
        return g

    g = lax.fori_loop(0, SWEEPS, sweep, g)
    sv = [jnp.sqrt(jnp.maximum(g[(k, k)], 0.0)) for k in range(K)]
    smax = functools.reduce(jnp.maximum, sv)
    ssum = functools.reduce(jnp.add, sv)
    return 1.0 - smax / (ssum + 1e-8)


def _k34_body(ps_ref, zb_ref, zf_ref, rb_ref, rf_ref, rec_ref, out_ref,
              curv_s, acc_s):
    i = pl.program_id(0)
    lane = lax.broadcasted_iota(jnp.int32, (1, 128), 1)

    @pl.when(i == 0)
    def _():
        curv_s[...] = _curvature(ps_ref)
        acc_s[...] = jnp.zeros((1, 128), jnp.float32)

    zb = zb_ref[...]
    zf = zf_ref[...]
    rb = rb_ref[...]
    rf = rf_ref[...]

    def dist(ab, af):
        p = lax.dot_general(ab, af, (((1,), (1,)), ((), ())), precision=HP)
        sq_b = jnp.sum(ab * ab, axis=1, keepdims=True)
        ones_row = jnp.ones((1, L), jnp.float32)
        sq_all = lax.dot_general(ones_row, af * af, (((1,), (1,)), ((), ())),
                                 precision=HP)
        return jnp.sqrt(jnp.maximum(sq_b + sq_all - 2.0 * p, 1e-12))

    zd = dist(zb, zf)
    rd = dist(rb, rf)
    crow = curv_s[pl.ds(i, 1), :]
    rowid = lax.broadcasted_iota(jnp.int32, (BLK, 128), 0)
    colid = lax.broadcasted_iota(jnp.int32, (BLK, 128), 1)
    ident = (rowid == colid).astype(jnp.float32)
    ca = lax.dot_general(ident, crow, (((1,), (1,)), ((), ())),
                         precision=HP)
    wchunks = [jnp.maximum(ca, curv_s[pl.ds(s, 1), :]) for s in range(SB)]
    cmax = jnp.concatenate(wchunks, axis=1)
    wgt = jnp.maximum(1.0 - cmax, 0.1)
    sa = jnp.sum(wgt * zd * zd)
    sb = jnp.sum(wgt * zd * rd)
    sc = jnp.sum(wgt * rd * rd)
    mz = jnp.max(zd)
    mr = jnp.max(rd)
    new = (sa * (lane == 0) + sb * (lane == 1) + sc * (lane == 2)
           + mz * (lane == 3) + mr * (lane == 4)).astype(jnp.float32)
    old = acc_s[...]
    acc_s[...] = jnp.where(lane < 3, old + new, jnp.maximum(old, new))

    @pl.when(i == NBLK - 1)
    def _():
        accv = acc_s[...]
        def pick(j):
            return jnp.sum(jnp.where(lane == j, accv, 0.0))
        a, bsum, c = pick(0), pick(1), pick(2)
        mzf = pick(3) + 1e-8
        mrf = pick(4) + 1e-8
        rec = jnp.sum(jnp.where(lane == 0, rec_ref[...], 0.0)) / (B * D)
        npairs = B * (B - 1)
        dist_loss = (a / (mzf * mzf) - 2.0 * bsum / (mzf * mrf)
                     + c / (mrf * mrf)) / npairs
        total = rec + dist_loss
        out_ref[...] = (total * (lane == 0) + rec * (lane == 1)
                        + dist_loss * (lane == 2)).astype(jnp.float32)


def _run_k34(psub3, z, ref_emb, rec_sum, interpret=False):
    full = lambda shape: pl.BlockSpec(shape, lambda i: (0,) * len(shape))
    return pl.pallas_call(
        _k34_body, grid=(NBLK,),
        in_specs=[
            full((NPAIR, SB, 128)),
            pl.BlockSpec((BLK, L), lambda i: (i, 0)), full((B, L)),
            pl.BlockSpec((BLK, L), lambda i: (i, 0)), full((B, L)),
            full((1, 128)),
        ],
        out_specs=pl.BlockSpec((1, 128), lambda i: (0, 0)),
        out_shape=jax.ShapeDtypeStruct((1, 128), jnp.float32),
        scratch_shapes=[
            pltpu.VMEM((SB, 128), jnp.float32),
            pltpu.VMEM((1, 128), jnp.float32),
        ],
        interpret=interpret,
    )(psub3, z, z, ref_emb, ref_emb, rec_sum)


def kernel(x, ref_emb, We1, be1, We2, be2, Wd1, bd1, Wd2, bd2):
    p, pairs_t = _run_kp(x)
    psub = _sc_gather(p.reshape(-1), pairs_t.reshape(-1))
    z, rec_sum = _run_kae(x, We1, be1, We2, be2, Wd1, bd1, Wd2, bd2)
    out = _run_k34(psub.reshape(NPAIR, SB, 128), z, ref_emb, rec_sum)
    return out[0, 0], out[0, 1], out[0, 2]


# speedup vs baseline: 288.7990x; 1.0769x over previous
import functools

import jax
import jax.numpy as jnp
from jax import lax
from jax.experimental import pallas as pl
from jax.experimental.pallas import tpu as pltpu
from jax.experimental.pallas import tpu_sc as plsc

B = 1024
D = 768
H = 512
L = 128
K = 10
BLK = 256
NBLK = B // BLK
SB = B // 128
UPAIRS = [(a, b) for a in range(K) for b in range(a, K)]
NPAIR = len(UPAIRS)
SWEEPS = 4
HP = jax.lax.Precision.DEFAULT


def _kp_body(xb_ref, xf_ref, p_ref, pair_ref, xsq_s):
    i = pl.program_id(0)
    xb = xb_ref[...]
    xf = xf_ref[...]

    @pl.when(i == 0)
    def _():
        xsq_s[...] = jnp.sum(xf * xf, axis=1, keepdims=True)

    pb = lax.dot_general(xf, xb, (((1,), (1,)), ((), ())), precision=HP)
    p_ref[...] = pb
    xsq_col = xsq_s[...]
    ones_row = jnp.ones((1, D), jnp.float32)
    xsq_row = lax.dot_general(ones_row, xb * xb, (((1,), (1,)), ((), ())),
                              precision=HP)
    d2 = jnp.maximum(xsq_col + xsq_row - 2.0 * pb, 1e-12)
    row = lax.broadcasted_iota(jnp.int32, (B, BLK), 0)
    col = lax.broadcasted_iota(jnp.int32, (B, BLK), 1)
    big = jnp.int32(2**31 - 1)
    key = ((lax.bitcast_convert_type(d2, jnp.int32) >> 10) << 10) | row
    key = jnp.where(row == col + i * BLK, big, key)
    idxs = []
    for _ in range(K):
        kmin = jnp.min(key, axis=0, keepdims=True)
        idxs.append(kmin & 1023)
        key = jnp.where(key == kmin, big, key)
    knn_t = jnp.concatenate(idxs, axis=0)
    knn_tb = knn_t * B
    parts = [knn_tb[a:a + 1, :] + knn_t[b:b + 1, :] for a, b in UPAIRS]
    pair_ref[...] = jnp.concatenate(parts, axis=0)


def _run_kp(x, interpret=False):
    full = lambda shape: pl.BlockSpec(shape, lambda i: (0,) * len(shape))
    return pl.pallas_call(
        _kp_body, grid=(NBLK,),
        in_specs=[pl.BlockSpec((BLK, D), lambda i: (i, 0)), full((B, D))],
        out_specs=(
            pl.BlockSpec((B, BLK), lambda i: (0, i)),
            pl.BlockSpec((NPAIR, BLK), lambda i: (0, i)),
        ),
        out_shape=(
            jax.ShapeDtypeStruct((B, B), jnp.float32),
            jax.ShapeDtypeStruct((NPAIR, B), jnp.int32),
        ),
        scratch_shapes=[pltpu.VMEM((B, 1), jnp.float32)],
        interpret=interpret,
    )(x, x)


def _kae_body(x_ref, we1_ref, be1_ref, we2_ref, be2_ref,
              wd1_ref, bd1_ref, wd2_ref, bd2_ref, z_ref, rec_ref):
    x = x_ref[...]
    h1 = jnp.maximum(jnp.dot(x, we1_ref[...], precision=HP) + be1_ref[...], 0.0)
    zb = jnp.dot(h1, we2_ref[...], precision=HP) + be2_ref[...]
    h2 = jnp.maximum(jnp.dot(zb, wd1_ref[...], precision=HP) + bd1_ref[...], 0.0)
    xr = jnp.dot(h2, wd2_ref[...], precision=HP) + bd2_ref[...]
    z_ref[...] = zb
    rec_ref[...] = jnp.full((1, 128), jnp.sum((xr - x) ** 2), jnp.float32)


def _run_kae(x, We1, be1, We2, be2, Wd1, bd1, Wd2, bd2, interpret=False):
    full = lambda shape: pl.BlockSpec(shape, lambda i: (0,) * len(shape))
    return pl.pallas_call(
        _kae_body, grid=(1,),
        in_specs=[
            full((B, D)),
            full((D, H)), full((1, H)),
            full((H, L)), full((1, L)),
            full((L, H)), full((1, H)),
            full((H, D)), full((1, D)),
        ],
        out_specs=(
            pl.BlockSpec((B, L), lambda i: (0, 0)),
            pl.BlockSpec((1, 128), lambda i: (0, 0)),
        ),
        out_shape=(
            jax.ShapeDtypeStruct((B, L), jnp.float32),
            jax.ShapeDtypeStruct((1, 128), jnp.float32),
        ),
        interpret=interpret,
    )(x, We1, be1.reshape(1, H), We2, be2.reshape(1, L),
      Wd1, bd1.reshape(1, H), Wd2, bd2.reshape(1, D))


def _sc_gather(p_flat, idx_flat):
    n = idx_flat.shape[0]
    info = plsc.get_sparse_core_info()
    nw = info.num_cores * info.num_subcores
    chunk = n // nw
    mesh = plsc.VectorSubcoreMesh(core_axis_name="c", subcore_axis_name="s")

    @functools.partial(
        pl.kernel, mesh=mesh,
        out_type=jax.ShapeDtypeStruct((n,), jnp.float32),
        scratch_types=[
            pltpu.VMEM((chunk,), jnp.int32),
            pltpu.VMEM((chunk,), jnp.float32),
            pltpu.SemaphoreType.DMA,
        ],
    )
    def gather_k(p_hbm, idx_hbm, out_hbm, idx_v, val_v, sem):
        wid = lax.axis_index("s") * info.num_cores + lax.axis_index("c")
        base = wid * chunk
        pltpu.sync_copy(idx_hbm.at[pl.ds(base, chunk)], idx_v)
        pltpu.async_copy(p_hbm.at[idx_v], val_v, sem).wait()
        pltpu.sync_copy(val_v, out_hbm.at[pl.ds(base, chunk)])

    return gather_k(p_flat, idx_flat)


def _curvature(ps_ref):
    ent = {ab: ps_ref[r] for r, ab in enumerate(UPAIRS)}

    def sym(a, b):
        return ent[(a, b)] if a <= b else ent[(b, a)]

    ra = [functools.reduce(jnp.add, [sym(a, b) for b in range(K)])
          for a in range(K)]
    s = functools.reduce(jnp.add, ra)
    sc2 = s / (K * K)
    g = {}
    for a in range(K):
        for b in range(a, K):
            g[(a, b)] = ent[(a, b)] - ra[a] / K - ra[b] / K + sc2

    def sweep(_, g):
        def get(a, b):
            return g[(a, b)] if a <= b else g[(b, a)]

        for p in range(K - 1):
            for q in range(p + 1, K):
                app = g[(p, p)]
                apq = g[(p, q)]
                aqq = g[(q, q)]
                small = jnp.abs(apq) < 1e-30
                safe = jnp.where(small, jnp.float32(1.0), apq)
                tau = (aqq - app) / (2.0 * safe)
                t = jnp.sign(tau) / (jnp.abs(tau) + jnp.sqrt(1.0 + tau * tau))
                c = 1.0 / jnp.sqrt(1.0 + t * t)
                sn = t * c
                t = jnp.where(small, jnp.float32(0.0), t)
                c = jnp.where(small, jnp.float32(1.0), c)
                sn = jnp.where(small, jnp.float32(0.0), sn)
                for k in range(K):
                    if k == p or k == q:
                        continue
                    akp = get(k, p)
                    akq = get(k, q)
                    nkp = c * akp - sn * akq
                    nkq = sn * akp + c * akq
                    if k <= p:
                        g[(k, p)] = nkp
                    else:
                        g[(p, k)] = nkp
                    if k <= q:
                        g[(k, q)] = nkq
                    else:
                        g[(q, k)] = nkq
                g[(p, p)] = app - t * apq
                g[(q, q)] = aqq + t * apq
                g[(p, q)] = jnp.zeros_like(apq)
        return g

    g = lax.fori_loop(0, SWEEPS, sweep, g)
    sv = [jnp.sqrt(jnp.maximum(g[(k, k)], 0.0)) for k in range(K)]
    smax = functools.reduce(jnp.maximum, sv)
    ssum = functools.reduce(jnp.add, sv)
    return 1.0 - smax / (ssum + 1e-8)


def _k34_body(ps_ref, zb_ref, zf_ref, rb_ref, rf_ref, rec_ref, out_ref,
              curv_s, acc_s):
    i = pl.program_id(0)
    lane = lax.broadcasted_iota(jnp.int32, (1, 128), 1)

    @pl.when(i == 0)
    def _():
        curv_s[...] = _curvature(ps_ref)
        acc_s[...] = jnp.zeros((1, 128), jnp.float32)

    zb = zb_ref[...]
    zf = zf_ref[...]
    rb = rb_ref[...]
    rf = rf_ref[...]

    def dist(ab, af):
        p = lax.dot_general(ab, af, (((1,), (1,)), ((), ())), precision=HP)
        sq_b = jnp.sum(ab * ab, axis=1, keepdims=True)
        ones_row = jnp.ones((1, L), jnp.float32)
        sq_all = lax.dot_general(ones_row, af * af, (((1,), (1,)), ((), ())),
                                 precision=HP)
        return jnp.sqrt(jnp.maximum(sq_b + sq_all - 2.0 * p, 1e-12))

    zd = dist(zb, zf)
    rd = dist(rb, rf)
    rowid = lax.broadcasted_iota(jnp.int32, (128, 128), 0)
    colid = lax.broadcasted_iota(jnp.int32, (128, 128), 1)
    ident = (rowid == colid).astype(jnp.float32)
    spb = BLK // 128
    ca = jnp.concatenate(
        [lax.dot_general(ident, curv_s[pl.ds(i * spb + u, 1), :],
                         (((1,), (1,)), ((), ())), precision=HP)
         for u in range(spb)], axis=0)
    wchunks = [jnp.maximum(ca, curv_s[pl.ds(s, 1), :]) for s in range(SB)]
    cmax = jnp.concatenate(wchunks, axis=1)
    wgt = jnp.maximum(1.0 - cmax, 0.1)
    sa = jnp.sum(wgt * zd * zd)
    sb = jnp.sum(wgt * zd * rd)
    sc = jnp.sum(wgt * rd * rd)
    mz = jnp.max(zd)
    mr = jnp.max(rd)
    new = (sa * (lane == 0) + sb * (lane == 1) + sc * (lane == 2)
           + mz * (lane == 3) + mr * (lane == 4)).astype(jnp.float32)
    old = acc_s[...]
    acc_s[...] = jnp.where(lane < 3, old + new, jnp.maximum(old, new))

    @pl.when(i == NBLK - 1)
    def _():
        accv = acc_s[...]
        def pick(j):
            return jnp.sum(jnp.where(lane == j, accv, 0.0))
        a, bsum, c = pick(0), pick(1), pick(2)
        mzf = pick(3) + 1e-8
        mrf = pick(4) + 1e-8
        rec = jnp.sum(jnp.where(lane == 0, rec_ref[...], 0.0)) / (B * D)
        npairs = B * (B - 1)
        dist_loss = (a / (mzf * mzf) - 2.0 * bsum / (mzf * mrf)
                     + c / (mrf * mrf)) / npairs
        total = rec + dist_loss
        out_ref[...] = (total * (lane == 0) + rec * (lane == 1)
                        + dist_loss * (lane == 2)).astype(jnp.float32)


def _run_k34(psub3, z, ref_emb, rec_sum, interpret=False):
    full = lambda shape: pl.BlockSpec(shape, lambda i: (0,) * len(shape))
    return pl.pallas_call(
        _k34_body, grid=(NBLK,),
        in_specs=[
            full((NPAIR, SB, 128)),
            pl.BlockSpec((BLK, L), lambda i: (i, 0)), full((B, L)),
            pl.BlockSpec((BLK, L), lambda i: (i, 0)), full((B, L)),
            full((1, 128)),
        ],
        out_specs=pl.BlockSpec((1, 128), lambda i: (0, 0)),
        out_shape=jax.ShapeDtypeStruct((1, 128), jnp.float32),
        scratch_shapes=[
            pltpu.VMEM((SB, 128), jnp.float32),
            pltpu.VMEM((1, 128), jnp.float32),
        ],
        interpret=interpret,
    )(psub3, z, z, ref_emb, ref_emb, rec_sum)


def kernel(x, ref_emb, We1, be1, We2, be2, Wd1, bd1, Wd2, bd2):
    p, pairs_t = _run_kp(x)
    psub = _sc_gather(p.reshape(-1), pairs_t.reshape(-1))
    z, rec_sum = _run_kae(x, We1, be1, We2, be2, Wd1, bd1, Wd2, bd2)
    out = _run_k34(psub.reshape(NPAIR, SB, 128), z, ref_emb, rec_sum)
    return out[0, 0], out[0, 1], out[0, 2]


# speedup vs baseline: 290.8459x; 1.0071x over previous
import functools

import jax
import jax.numpy as jnp
from jax import lax
from jax.experimental import pallas as pl
from jax.experimental.pallas import tpu as pltpu
from jax.experimental.pallas import tpu_sc as plsc

B = 1024
D = 768
H = 512
L = 128
K = 10
BLK = 512
NBLK = B // BLK
SB = B // 128
UPAIRS = [(a, b) for a in range(K) for b in range(a, K)]
NPAIR = len(UPAIRS)
SWEEPS = 4
HP = jax.lax.Precision.DEFAULT


def _kp_body(xb_ref, xf_ref, p_ref, pair_ref, xsq_s):
    i = pl.program_id(0)
    xb = xb_ref[...]
    xf = xf_ref[...]

    @pl.when(i == 0)
    def _():
        xsq_s[...] = jnp.sum(xf * xf, axis=1, keepdims=True)

    pb = lax.dot_general(xf, xb, (((1,), (1,)), ((), ())), precision=HP)
    p_ref[...] = pb
    xsq_col = xsq_s[...]
    ones_row = jnp.ones((1, D), jnp.float32)
    xsq_row = lax.dot_general(ones_row, xb * xb, (((1,), (1,)), ((), ())),
                              precision=HP)
    d2 = jnp.maximum(xsq_col + xsq_row - 2.0 * pb, 1e-12)
    row = lax.broadcasted_iota(jnp.int32, (B, BLK), 0)
    col = lax.broadcasted_iota(jnp.int32, (B, BLK), 1)
    big = jnp.int32(2**31 - 1)
    key = ((lax.bitcast_convert_type(d2, jnp.int32) >> 10) << 10) | row
    key = jnp.where(row == col + i * BLK, big, key)
    idxs = []
    for _ in range(K):
        kmin = jnp.min(key, axis=0, keepdims=True)
        idxs.append(kmin & 1023)
        key = jnp.where(key == kmin, big, key)
    knn_t = jnp.concatenate(idxs, axis=0)
    knn_tb = knn_t * B
    parts = [knn_tb[a:a + 1, :] + knn_t[b:b + 1, :] for a, b in UPAIRS]
    pair_ref[...] = jnp.concatenate(parts, axis=0)


def _run_kp(x, interpret=False):
    full = lambda shape: pl.BlockSpec(shape, lambda i: (0,) * len(shape))
    return pl.pallas_call(
        _kp_body, grid=(NBLK,),
        in_specs=[pl.BlockSpec((BLK, D), lambda i: (i, 0)), full((B, D))],
        out_specs=(
            pl.BlockSpec((B, BLK), lambda i: (0, i)),
            pl.BlockSpec((NPAIR, BLK), lambda i: (0, i)),
        ),
        out_shape=(
            jax.ShapeDtypeStruct((B, B), jnp.float32),
            jax.ShapeDtypeStruct((NPAIR, B), jnp.int32),
        ),
        scratch_shapes=[pltpu.VMEM((B, 1), jnp.float32)],
        interpret=interpret,
    )(x, x)


def _kae_body(x_ref, we1_ref, be1_ref, we2_ref, be2_ref,
              wd1_ref, bd1_ref, wd2_ref, bd2_ref, z_ref, rec_ref):
    x = x_ref[...]
    h1 = jnp.maximum(jnp.dot(x, we1_ref[...], precision=HP) + be1_ref[...], 0.0)
    zb = jnp.dot(h1, we2_ref[...], precision=HP) + be2_ref[...]
    h2 = jnp.maximum(jnp.dot(zb, wd1_ref[...], precision=HP) + bd1_ref[...], 0.0)
    xr = jnp.dot(h2, wd2_ref[...], precision=HP) + bd2_ref[...]
    z_ref[...] = zb
    rec_ref[...] = jnp.full((1, 128), jnp.sum((xr - x) ** 2), jnp.float32)


def _run_kae(x, We1, be1, We2, be2, Wd1, bd1, Wd2, bd2, interpret=False):
    full = lambda shape: pl.BlockSpec(shape, lambda i: (0,) * len(shape))
    return pl.pallas_call(
        _kae_body, grid=(1,),
        in_specs=[
            full((B, D)),
            full((D, H)), full((1, H)),
            full((H, L)), full((1, L)),
            full((L, H)), full((1, H)),
            full((H, D)), full((1, D)),
        ],
        out_specs=(
            pl.BlockSpec((B, L), lambda i: (0, 0)),
            pl.BlockSpec((1, 128), lambda i: (0, 0)),
        ),
        out_shape=(
            jax.ShapeDtypeStruct((B, L), jnp.float32),
            jax.ShapeDtypeStruct((1, 128), jnp.float32),
        ),
        interpret=interpret,
    )(x, We1, be1.reshape(1, H), We2, be2.reshape(1, L),
      Wd1, bd1.reshape(1, H), Wd2, bd2.reshape(1, D))


def _sc_gather(p_flat, idx_flat):
    n = idx_flat.shape[0]
    info = plsc.get_sparse_core_info()
    nw = info.num_cores * info.num_subcores
    chunk = n // nw
    mesh = plsc.VectorSubcoreMesh(core_axis_name="c", subcore_axis_name="s")

    @functools.partial(
        pl.kernel, mesh=mesh,
        out_type=jax.ShapeDtypeStruct((n,), jnp.float32),
        scratch_types=[
            pltpu.VMEM((chunk,), jnp.int32),
            pltpu.VMEM((chunk,), jnp.float32),
            pltpu.SemaphoreType.DMA,
        ],
    )
    def gather_k(p_hbm, idx_hbm, out_hbm, idx_v, val_v, sem):
        wid = lax.axis_index("s") * info.num_cores + lax.axis_index("c")
        base = wid * chunk
        pltpu.sync_copy(idx_hbm.at[pl.ds(base, chunk)], idx_v)
        pltpu.async_copy(p_hbm.at[idx_v], val_v, sem).wait()
        pltpu.sync_copy(val_v, out_hbm.at[pl.ds(base, chunk)])

    return gather_k(p_flat, idx_flat)


def _curvature(ps_ref):
    ent = {ab: ps_ref[r] for r, ab in enumerate(UPAIRS)}

    def sym(a, b):
        return ent[(a, b)] if a <= b else ent[(b, a)]

    ra = [functools.reduce(jnp.add, [sym(a, b) for b in range(K)])
          for a in range(K)]
    s = functools.reduce(jnp.add, ra)
    sc2 = s / (K * K)
    g = {}
    for a in range(K):
        for b in range(a, K):
            g[(a, b)] = ent[(a, b)] - ra[a] / K - ra[b] / K + sc2

    def sweep(_, g):
        def get(a, b):
            return g[(a, b)] if a <= b else g[(b, a)]

        for p in range(K - 1):
            for q in range(p + 1, K):
                app = g[(p, p)]
                apq = g[(p, q)]
                aqq = g[(q, q)]
                small = jnp.abs(apq) < 1e-30
                safe = jnp.where(small, jnp.float32(1.0), apq)
                tau = (aqq - app) / (2.0 * safe)
                t = jnp.sign(tau) / (jnp.abs(tau) + jnp.sqrt(1.0 + tau * tau))
                c = 1.0 / jnp.sqrt(1.0 + t * t)
                sn = t * c
                t = jnp.where(small, jnp.float32(0.0), t)
                c = jnp.where(small, jnp.float32(1.0), c)
                sn = jnp.where(small, jnp.float32(0.0), sn)
                for k in range(K):
                    if k == p or k == q:
                        continue
                    akp = get(k, p)
                    akq = get(k, q)
                    nkp = c * akp - sn * akq
                    nkq = sn * akp + c * akq
                    if k <= p:
                        g[(k, p)] = nkp
                    else:
                        g[(p, k)] = nkp
                    if k <= q:
                        g[(k, q)] = nkq
                    else:
                        g[(q, k)] = nkq
                g[(p, p)] = app - t * apq
                g[(q, q)] = aqq + t * apq
                g[(p, q)] = jnp.zeros_like(apq)
        return g

    g = lax.fori_loop(0, SWEEPS, sweep, g)
    sv = [jnp.sqrt(jnp.maximum(g[(k, k)], 0.0)) for k in range(K)]
    smax = functools.reduce(jnp.maximum, sv)
    ssum = functools.reduce(jnp.add, sv)
    return 1.0 - smax / (ssum + 1e-8)


def _k34_body(ps_ref, zb_ref, zf_ref, rb_ref, rf_ref, rec_ref, out_ref,
              curv_s, acc_s):
    i = pl.program_id(0)
    lane = lax.broadcasted_iota(jnp.int32, (1, 128), 1)

    @pl.when(i == 0)
    def _():
        curv_s[...] = _curvature(ps_ref)
        acc_s[...] = jnp.zeros((1, 128), jnp.float32)

    zb = zb_ref[...]
    zf = zf_ref[...]
    rb = rb_ref[...]
    rf = rf_ref[...]

    def dist(ab, af):
        p = lax.dot_general(ab, af, (((1,), (1,)), ((), ())), precision=HP)
        sq_b = jnp.sum(ab * ab, axis=1, keepdims=True)
        ones_row = jnp.ones((1, L), jnp.float32)
        sq_all = lax.dot_general(ones_row, af * af, (((1,), (1,)), ((), ())),
                                 precision=HP)
        return jnp.sqrt(jnp.maximum(sq_b + sq_all - 2.0 * p, 1e-12))

    zd = dist(zb, zf)
    rd = dist(rb, rf)
    rowid = lax.broadcasted_iota(jnp.int32, (128, 128), 0)
    colid = lax.broadcasted_iota(jnp.int32, (128, 128), 1)
    ident = (rowid == colid).astype(jnp.float32)
    spb = BLK // 128
    ca = jnp.concatenate(
        [lax.dot_general(ident, curv_s[pl.ds(i * spb + u, 1), :],
                         (((1,), (1,)), ((), ())), precision=HP)
         for u in range(spb)], axis=0)
    wchunks = [jnp.maximum(ca, curv_s[pl.ds(s, 1), :]) for s in range(SB)]
    cmax = jnp.concatenate(wchunks, axis=1)
    wgt = jnp.maximum(1.0 - cmax, 0.1)
    sa = jnp.sum(wgt * zd * zd)
    sb = jnp.sum(wgt * zd * rd)
    sc = jnp.sum(wgt * rd * rd)
    mz = jnp.max(zd)
    mr = jnp.max(rd)
    new = (sa * (lane == 0) + sb * (lane == 1) + sc * (lane == 2)
           + mz * (lane == 3) + mr * (lane == 4)).astype(jnp.float32)
    old = acc_s[...]
    acc_s[...] = jnp.where(lane < 3, old + new, jnp.maximum(old, new))

    @pl.when(i == NBLK - 1)
    def _():
        accv = acc_s[...]
        def pick(j):
            return jnp.sum(jnp.where(lane == j, accv, 0.0))
        a, bsum, c = pick(0), pick(1), pick(2)
        mzf = pick(3) + 1e-8
        mrf = pick(4) + 1e-8
        rec = jnp.sum(jnp.where(lane == 0, rec_ref[...], 0.0)) / (B * D)
        npairs = B * (B - 1)
        dist_loss = (a / (mzf * mzf) - 2.0 * bsum / (mzf * mrf)
                     + c / (mrf * mrf)) / npairs
        total = rec + dist_loss
        out_ref[...] = (total * (lane == 0) + rec * (lane == 1)
                        + dist_loss * (lane == 2)).astype(jnp.float32)


def _run_k34(psub3, z, ref_emb, rec_sum, interpret=False):
    full = lambda shape: pl.BlockSpec(shape, lambda i: (0,) * len(shape))
    return pl.pallas_call(
        _k34_body, grid=(NBLK,),
        in_specs=[
            full((NPAIR, SB, 128)),
            pl.BlockSpec((BLK, L), lambda i: (i, 0)), full((B, L)),
            pl.BlockSpec((BLK, L), lambda i: (i, 0)), full((B, L)),
            full((1, 128)),
        ],
        out_specs=pl.BlockSpec((1, 128), lambda i: (0, 0)),
        out_shape=jax.ShapeDtypeStruct((1, 128), jnp.float32),
        scratch_shapes=[
            pltpu.VMEM((SB, 128), jnp.float32),
            pltpu.VMEM((1, 128), jnp.float32),
        ],
        interpret=interpret,
    )(psub3, z, z, ref_emb, ref_emb, rec_sum)


def kernel(x, ref_emb, We1, be1, We2, be2, Wd1, bd1, Wd2, bd2):
    p, pairs_t = _run_kp(x)
    psub = _sc_gather(p.reshape(-1), pairs_t.reshape(-1))
    z, rec_sum = _run_kae(x, We1, be1, We2, be2, Wd1, bd1, Wd2, bd2)
    out = _run_k34(psub.reshape(NPAIR, SB, 128), z, ref_emb, rec_sum)
    return out[0, 0], out[0, 1], out[0, 2]
